# baseline (speedup 1.0000x reference)
import jax
import jax.numpy as jnp
from jax import lax
from jax.experimental import pallas as pl
from jax.experimental.pallas import tpu as pltpu
from jax.experimental.pallas import tpu_sc as plsc

B = 16384
H = 128
NC = 2
NS = 16
NW = NC * NS
BPW = B // NW
CHUNK = 128
NCHUNK = BPW // CHUNK
NVEC = H // 16

BM = 2048


def _gather_body(pt_hbm, ct_hbm, bt_hbm, ip_hbm, ic_hbm, ib_hbm,
                 out_hbm,
                 idxp, idxc, idxb, acc, ob, ctab_s, btab_s,
                 isem, psem, bsem0, bsem1, osem):
    wid = lax.axis_index("s") * NC + lax.axis_index("c")
    base = wid * BPW

    idescs = []
    for src, dst in ((ip_hbm, idxp), (ic_hbm, idxc), (ib_hbm, idxb)):
        for j in range(NCHUNK):
            idescs.append(pltpu.async_copy(
                src.at[pl.ds(base + j * CHUNK, CHUNK)], dst.at[j], isem))

    @pl.when(lax.axis_index("s") == 0)
    def _():
        pltpu.sync_copy(ct_hbm, ctab_s)
        pltpu.sync_copy(bt_hbm, btab_s)

    for d in idescs:
        d.wait()

    pdescs = [
        pltpu.async_copy(pt_hbm.at[idxp.at[j]],
                         acc.at[pl.ds(j * CHUNK, CHUNK)], psem)
        for j in range(NCHUNK)
    ]

    plsc.subcore_barrier()

    cdescs = {}
    for j in range(NCHUNK):
        pdescs[j].wait()
        cdescs[j] = pltpu.async_copy(
            ctab_s.at[idxc.at[j]], acc.at[pl.ds(j * CHUNK, CHUNK)], bsem0,
            add=True)
    bdescs = {}
    for j in range(NCHUNK):
        cdescs[j].wait()
        bdescs[j] = pltpu.async_copy(
            btab_s.at[idxb.at[j]], acc.at[pl.ds(j * CHUNK, CHUNK)], bsem1,
            add=True)
    wdescs = []
    for j in range(NCHUNK):
        bdescs[j].wait()
        if j >= 2:
            wdescs[j - 2].wait()
        cbase = j * CHUNK
        ko = j % 2

        def pack_body(i, _, ko=ko, cbase=cbase):
            for q in range(H // 32):
                a = acc[cbase + i, pl.ds(32 * q, 16)]
                b = acc[cbase + i, pl.ds(32 * q + 16, 16)]
                ob[ko, i, pl.ds(32 * q, 32)] = plsc.pack(
                    a, b, format=plsc.PackFormat.INTERLEAVED)
            return _

        lax.fori_loop(0, CHUNK, pack_body, None)
        wdescs.append(pltpu.async_copy(
            ob.at[ko], out_hbm.at[pl.ds(base + cbase, CHUNK)], osem))
    for j in range(max(0, NCHUNK - 2), NCHUNK):
        wdescs[j].wait()


_gather = pl.kernel(
    _gather_body,
    out_type=jax.ShapeDtypeStruct((B, H), jnp.bfloat16),
    mesh=plsc.VectorSubcoreMesh(core_axis_name="c", subcore_axis_name="s",
                                num_cores=NC, num_subcores=NS),
    compiler_params=pltpu.CompilerParams(needs_layout_passes=False),
    scratch_types=[
        pltpu.VMEM((NCHUNK, CHUNK), jnp.int32),
        pltpu.VMEM((NCHUNK, CHUNK), jnp.int32),
        pltpu.VMEM((NCHUNK, CHUNK), jnp.int32),
        pltpu.VMEM((BPW, H), jnp.float32),
        pltpu.VMEM((2, CHUNK, H), jnp.bfloat16),
        pltpu.VMEM_SHARED((1000, H), jnp.float32),
        pltpu.VMEM_SHARED((1000, H), jnp.float32),
        pltpu.SemaphoreType.DMA,
        pltpu.SemaphoreType.DMA,
        pltpu.SemaphoreType.DMA,
        pltpu.SemaphoreType.DMA,
        pltpu.SemaphoreType.DMA,
    ],
)


def _mlp_body(h_ref, w1_ref, b1_ref, w2_ref, b2_ref, out_ref):
    h = h_ref[...]
    a = jnp.dot(h, w1_ref[...].astype(jnp.bfloat16),
                preferred_element_type=jnp.float32) + b1_ref[...]
    a = a * jax.nn.sigmoid(a)
    out_ref[...] = (jnp.dot(a, w2_ref[...], preferred_element_type=jnp.float32)
                    + b2_ref[...])


def _mlp(hidden, W1, b1, W2, b2):
    grid = (B // BM,)
    row_spec = pl.BlockSpec((BM, H), lambda i: (i, 0))
    full = pl.BlockSpec((H, H), lambda i: (0, 0))
    bias = pl.BlockSpec((1, H), lambda i: (0, 0))
    return pl.pallas_call(
        _mlp_body,
        grid=grid,
        in_specs=[row_spec, full, bias, full, bias],
        out_specs=row_spec,
        out_shape=jax.ShapeDtypeStruct((B, H), jnp.float32),
    )(hidden, W1, b1.reshape(1, H), W2, b2.reshape(1, H))


def kernel(perturbation, cell_type, batch, perturb_table, cell_table,
           batch_table, W1, b1, W2, b2):
    ip = perturbation.astype(jnp.int32)
    ic = cell_type.astype(jnp.int32)
    ib = batch.astype(jnp.int32)
    hidden = _gather(perturb_table, cell_table, batch_table, ip, ic, ib)
    return _mlp(hidden, W1, b1, W2, b2)


# speedup vs baseline: 1.1130x; 1.1130x over previous
import jax
import jax.numpy as jnp
from jax import lax
from jax.experimental import pallas as pl
from jax.experimental.pallas import tpu as pltpu
from jax.experimental.pallas import tpu_sc as plsc

B = 16384
H = 128
NC = 2
NS = 16
NW = NC * NS
BPW = B // NW
CHUNK = 128
NCHUNK = BPW // CHUNK

BM = 2048


def _gather_body(pt_hbm, ct_hbm, bt_hbm, ip_hbm, ic_hbm, ib_hbm,
                 out_hbm,
                 idxp, idxc, idxb, acc, ctab_s, btab_s,
                 isem, psem, bsem0, bsem1, osem):
    wid = lax.axis_index("s") * NC + lax.axis_index("c")
    base = wid * BPW

    idescs = []
    for src, dst in ((ip_hbm, idxp), (ic_hbm, idxc), (ib_hbm, idxb)):
        for j in range(NCHUNK):
            idescs.append(pltpu.async_copy(
                src.at[pl.ds(base + j * CHUNK, CHUNK)], dst.at[j], isem))

    @pl.when(lax.axis_index("s") == 0)
    def _():
        pltpu.sync_copy(ct_hbm, ctab_s)
        pltpu.sync_copy(bt_hbm, btab_s)

    for d in idescs:
        d.wait()

    pdescs = [
        pltpu.async_copy(pt_hbm.at[idxp.at[j]],
                         acc.at[pl.ds(j * CHUNK, CHUNK)], psem)
        for j in range(NCHUNK)
    ]

    plsc.subcore_barrier()

    cdescs = {}
    for j in range(NCHUNK):
        pdescs[j].wait()
        cdescs[j] = pltpu.async_copy(
            ctab_s.at[idxc.at[j]], acc.at[pl.ds(j * CHUNK, CHUNK)], bsem0,
            add=True)
    bdescs = {}
    for j in range(NCHUNK):
        cdescs[j].wait()
        bdescs[j] = pltpu.async_copy(
            btab_s.at[idxb.at[j]], acc.at[pl.ds(j * CHUNK, CHUNK)], bsem1,
            add=True)
    wdescs = []
    for j in range(NCHUNK):
        bdescs[j].wait()
        wdescs.append(pltpu.async_copy(
            acc.at[pl.ds(j * CHUNK, CHUNK)],
            out_hbm.at[pl.ds(base + j * CHUNK, CHUNK)], osem))
    for d in wdescs:
        d.wait()


_gather = pl.kernel(
    _gather_body,
    out_type=jax.ShapeDtypeStruct((B, H), jnp.float32),
    mesh=plsc.VectorSubcoreMesh(core_axis_name="c", subcore_axis_name="s",
                                num_cores=NC, num_subcores=NS),
    scratch_types=[
        pltpu.VMEM((NCHUNK, CHUNK), jnp.int32),
        pltpu.VMEM((NCHUNK, CHUNK), jnp.int32),
        pltpu.VMEM((NCHUNK, CHUNK), jnp.int32),
        pltpu.VMEM((BPW, H), jnp.float32),
        pltpu.VMEM_SHARED((1000, H), jnp.float32),
        pltpu.VMEM_SHARED((1000, H), jnp.float32),
        pltpu.SemaphoreType.DMA,
        pltpu.SemaphoreType.DMA,
        pltpu.SemaphoreType.DMA,
        pltpu.SemaphoreType.DMA,
        pltpu.SemaphoreType.DMA,
    ],
)


def _mlp_body(h_ref, w1_ref, b1_ref, w2_ref, b2_ref, out_ref):
    h = h_ref[...]
    a = jnp.dot(h, w1_ref[...], preferred_element_type=jnp.float32) + b1_ref[...]
    a = a * jax.nn.sigmoid(a)
    out_ref[...] = (jnp.dot(a, w2_ref[...], preferred_element_type=jnp.float32)
                    + b2_ref[...])


def _mlp(hidden, W1, b1, W2, b2):
    grid = (B // BM,)
    row_spec = pl.BlockSpec((BM, H), lambda i: (i, 0))
    full = pl.BlockSpec((H, H), lambda i: (0, 0))
    bias = pl.BlockSpec((1, H), lambda i: (0, 0))
    return pl.pallas_call(
        _mlp_body,
        grid=grid,
        in_specs=[row_spec, full, bias, full, bias],
        out_specs=row_spec,
        out_shape=jax.ShapeDtypeStruct((B, H), jnp.float32),
    )(hidden, W1, b1.reshape(1, H), W2, b2.reshape(1, H))


def kernel(perturbation, cell_type, batch, perturb_table, cell_table,
           batch_table, W1, b1, W2, b2):
    ip = perturbation.astype(jnp.int32)
    ic = cell_type.astype(jnp.int32)
    ib = batch.astype(jnp.int32)
    hidden = _gather(perturb_table, cell_table, batch_table, ip, ic, ib)
    return _mlp(hidden, W1, b1, W2, b2)


# speedup vs baseline: 1.1636x; 1.0454x over previous
import jax
import jax.numpy as jnp
from jax import lax
from jax.experimental import pallas as pl
from jax.experimental.pallas import tpu as pltpu
from jax.experimental.pallas import tpu_sc as plsc

B = 16384
H = 128
NC = 2
NS = 16
NW = NC * NS
BPW = B // NW
CHUNK = 128
NCHUNK = BPW // CHUNK

BM = 4096


def _gather_body(pt_hbm, ct_hbm, bt_hbm, ip_hbm, ic_hbm, ib_hbm,
                 out_hbm,
                 idxp, idxc, idxb, acc, ctab_s, btab_s,
                 isem, psem, bsem0, bsem1, osem):
    wid = lax.axis_index("s") * NC + lax.axis_index("c")
    base = wid * BPW

    idescs = []
    for src, dst in ((ip_hbm, idxp), (ic_hbm, idxc), (ib_hbm, idxb)):
        for j in range(NCHUNK):
            idescs.append(pltpu.async_copy(
                src.at[pl.ds(base + j * CHUNK, CHUNK)], dst.at[j], isem))

    @pl.when(lax.axis_index("s") == 0)
    def _():
        pltpu.sync_copy(ct_hbm, ctab_s)
        pltpu.sync_copy(bt_hbm, btab_s)

    for d in idescs:
        d.wait()

    pdescs = [
        pltpu.async_copy(pt_hbm.at[idxp.at[j]],
                         acc.at[pl.ds(j * CHUNK, CHUNK)], psem)
        for j in range(NCHUNK)
    ]

    plsc.subcore_barrier()

    cdescs = {}
    for j in range(NCHUNK):
        pdescs[j].wait()
        cdescs[j] = pltpu.async_copy(
            ctab_s.at[idxc.at[j]], acc.at[pl.ds(j * CHUNK, CHUNK)], bsem0,
            add=True)
    bdescs = {}
    for j in range(NCHUNK):
        cdescs[j].wait()
        bdescs[j] = pltpu.async_copy(
            btab_s.at[idxb.at[j]], acc.at[pl.ds(j * CHUNK, CHUNK)], bsem1,
            add=True)
    wdescs = []
    for j in range(NCHUNK):
        bdescs[j].wait()
        wdescs.append(pltpu.async_copy(
            acc.at[pl.ds(j * CHUNK, CHUNK)],
            out_hbm.at[pl.ds(base + j * CHUNK, CHUNK)], osem))
    for d in wdescs:
        d.wait()


_gather = pl.kernel(
    _gather_body,
    out_type=jax.ShapeDtypeStruct((B, H), jnp.float32),
    mesh=plsc.VectorSubcoreMesh(core_axis_name="c", subcore_axis_name="s",
                                num_cores=NC, num_subcores=NS),
    scratch_types=[
        pltpu.VMEM((NCHUNK, CHUNK), jnp.int32),
        pltpu.VMEM((NCHUNK, CHUNK), jnp.int32),
        pltpu.VMEM((NCHUNK, CHUNK), jnp.int32),
        pltpu.VMEM((BPW, H), jnp.float32),
        pltpu.VMEM_SHARED((1000, H), jnp.float32),
        pltpu.VMEM_SHARED((1000, H), jnp.float32),
        pltpu.SemaphoreType.DMA,
        pltpu.SemaphoreType.DMA,
        pltpu.SemaphoreType.DMA,
        pltpu.SemaphoreType.DMA,
        pltpu.SemaphoreType.DMA,
    ],
)


def _mlp_body(h_ref, w1_ref, b1_ref, w2_ref, b2_ref, out_ref):
    h = h_ref[...]
    a = jnp.dot(h, w1_ref[...], preferred_element_type=jnp.float32) + b1_ref[...]
    a = a * jax.nn.sigmoid(a)
    out_ref[...] = (jnp.dot(a, w2_ref[...], preferred_element_type=jnp.float32)
                    + b2_ref[...])


def _mlp(hidden, W1, b1, W2, b2):
    grid = (B // BM,)
    row_spec = pl.BlockSpec((BM, H), lambda i: (i, 0))
    full = pl.BlockSpec((H, H), lambda i: (0, 0))
    bias = pl.BlockSpec((1, H), lambda i: (0, 0))
    return pl.pallas_call(
        _mlp_body,
        grid=grid,
        in_specs=[row_spec, full, bias, full, bias],
        out_specs=row_spec,
        out_shape=jax.ShapeDtypeStruct((B, H), jnp.float32),
    )(hidden, W1, b1.reshape(1, H), W2, b2.reshape(1, H))


def kernel(perturbation, cell_type, batch, perturb_table, cell_table,
           batch_table, W1, b1, W2, b2):
    ip = perturbation.astype(jnp.int32)
    ic = cell_type.astype(jnp.int32)
    ib = batch.astype(jnp.int32)
    hidden = _gather(perturb_table, cell_table, batch_table, ip, ic, ib)
    return _mlp(hidden, W1, b1, W2, b2)


# speedup vs baseline: 1.2119x; 1.0416x over previous
import jax
import jax.numpy as jnp
from jax import lax
from jax.experimental import pallas as pl
from jax.experimental.pallas import tpu as pltpu
from jax.experimental.pallas import tpu_sc as plsc

B = 16384
H = 128
NC = 2
NS = 16
NW = NC * NS
BPW = B // NW
CHUNK = 128
NCHUNK = BPW // CHUNK

BM = 8192


def _gather_body(pt_hbm, ct_hbm, bt_hbm, ip_hbm, ic_hbm, ib_hbm,
                 out_hbm,
                 idxp, idxc, idxb, acc, ctab_s, btab_s,
                 isem, psem, bsem0, bsem1, osem):
    wid = lax.axis_index("s") * NC + lax.axis_index("c")
    base = wid * BPW

    idescs = []
    for src, dst in ((ip_hbm, idxp), (ic_hbm, idxc), (ib_hbm, idxb)):
        for j in range(NCHUNK):
            idescs.append(pltpu.async_copy(
                src.at[pl.ds(base + j * CHUNK, CHUNK)], dst.at[j], isem))

    @pl.when(lax.axis_index("s") == 0)
    def _():
        pltpu.sync_copy(ct_hbm, ctab_s)
        pltpu.sync_copy(bt_hbm, btab_s)

    for d in idescs:
        d.wait()

    pdescs = [
        pltpu.async_copy(pt_hbm.at[idxp.at[j]],
                         acc.at[pl.ds(j * CHUNK, CHUNK)], psem)
        for j in range(NCHUNK)
    ]

    plsc.subcore_barrier()

    cdescs = {}
    for j in range(NCHUNK):
        pdescs[j].wait()
        cdescs[j] = pltpu.async_copy(
            ctab_s.at[idxc.at[j]], acc.at[pl.ds(j * CHUNK, CHUNK)], bsem0,
            add=True)
    bdescs = {}
    for j in range(NCHUNK):
        cdescs[j].wait()
        bdescs[j] = pltpu.async_copy(
            btab_s.at[idxb.at[j]], acc.at[pl.ds(j * CHUNK, CHUNK)], bsem1,
            add=True)
    wdescs = []
    for j in range(NCHUNK):
        bdescs[j].wait()
        wdescs.append(pltpu.async_copy(
            acc.at[pl.ds(j * CHUNK, CHUNK)],
            out_hbm.at[pl.ds(base + j * CHUNK, CHUNK)], osem))
    for d in wdescs:
        d.wait()


_gather = pl.kernel(
    _gather_body,
    out_type=jax.ShapeDtypeStruct((B, H), jnp.float32),
    mesh=plsc.VectorSubcoreMesh(core_axis_name="c", subcore_axis_name="s",
                                num_cores=NC, num_subcores=NS),
    scratch_types=[
        pltpu.VMEM((NCHUNK, CHUNK), jnp.int32),
        pltpu.VMEM((NCHUNK, CHUNK), jnp.int32),
        pltpu.VMEM((NCHUNK, CHUNK), jnp.int32),
        pltpu.VMEM((BPW, H), jnp.float32),
        pltpu.VMEM_SHARED((1000, H), jnp.float32),
        pltpu.VMEM_SHARED((1000, H), jnp.float32),
        pltpu.SemaphoreType.DMA,
        pltpu.SemaphoreType.DMA,
        pltpu.SemaphoreType.DMA,
        pltpu.SemaphoreType.DMA,
        pltpu.SemaphoreType.DMA,
    ],
)


def _mlp_body(h_ref, w1_ref, b1_ref, w2_ref, b2_ref, out_ref):
    h = h_ref[...]
    a = jnp.dot(h, w1_ref[...], preferred_element_type=jnp.float32) + b1_ref[...]
    a = a * jax.nn.sigmoid(a)
    out_ref[...] = (jnp.dot(a, w2_ref[...], preferred_element_type=jnp.float32)
                    + b2_ref[...])


def _mlp(hidden, W1, b1, W2, b2):
    grid = (B // BM,)
    row_spec = pl.BlockSpec((BM, H), lambda i: (i, 0))
    full = pl.BlockSpec((H, H), lambda i: (0, 0))
    bias = pl.BlockSpec((1, H), lambda i: (0, 0))
    return pl.pallas_call(
        _mlp_body,
        grid=grid,
        in_specs=[row_spec, full, bias, full, bias],
        out_specs=row_spec,
        out_shape=jax.ShapeDtypeStruct((B, H), jnp.float32),
    )(hidden, W1, b1.reshape(1, H), W2, b2.reshape(1, H))


def kernel(perturbation, cell_type, batch, perturb_table, cell_table,
           batch_table, W1, b1, W2, b2):
    ip = perturbation.astype(jnp.int32)
    ic = cell_type.astype(jnp.int32)
    ib = batch.astype(jnp.int32)
    hidden = _gather(perturb_table, cell_table, batch_table, ip, ic, ib)
    return _mlp(hidden, W1, b1, W2, b2)
